# baseline (speedup 1.0000x reference)
import jax
import jax.numpy as jnp
from jax import lax
from jax.experimental import pallas as pl

OUT_BLK = 256


def _body(x_ref, w_ref, p_ref, b_ref, o_ref):
    w = jnp.where(p_ref[...].astype(jnp.int32) != 0, w_ref[...], 0.0)
    acc = lax.dot_general(
        x_ref[...], w, (((1,), (1,)), ((), ())),
        preferred_element_type=jnp.float32,
    )
    o_ref[...] = acc + b_ref[...]


def kernel(x, W, bias, mask):
    orig_shape = x.shape
    in_features = W.shape[1]
    out_features = W.shape[0]
    x2 = x.reshape(-1, in_features)
    batch = x2.shape[0]
    bias2 = bias.reshape(1, out_features)
    packed = mask.astype(jnp.int4)
    y = pl.pallas_call(
        _body,
        grid=(out_features // OUT_BLK,),
        in_specs=[
            pl.BlockSpec((batch, in_features), lambda j: (0, 0)),
            pl.BlockSpec((OUT_BLK, in_features), lambda j: (j, 0)),
            pl.BlockSpec((OUT_BLK, in_features), lambda j: (j, 0)),
            pl.BlockSpec((1, OUT_BLK), lambda j: (0, j)),
        ],
        out_specs=pl.BlockSpec((batch, OUT_BLK), lambda j: (0, j)),
        out_shape=jax.ShapeDtypeStruct((batch, out_features), jnp.float32),
    )(x2, W, packed, bias2)
    return y.reshape(orig_shape[:-1] + (out_features,))


# speedup vs baseline: 1.1352x; 1.1352x over previous
import jax
import jax.numpy as jnp
from jax import lax
from jax.experimental import pallas as pl
from jax.experimental.pallas import tpu as pltpu

OUT_BLK = 256


def _body(x_ref, w_ref, p_ref, b_ref, o_ref):
    w = jnp.where(p_ref[...].astype(jnp.int32) != 0, w_ref[...], 0.0)
    acc = lax.dot_general(
        x_ref[...], w, (((1,), (1,)), ((), ())),
        preferred_element_type=jnp.float32,
    )
    o_ref[...] = acc + b_ref[...]


def kernel(x, W, bias, mask):
    orig_shape = x.shape
    in_features = W.shape[1]
    out_features = W.shape[0]
    x2 = x.reshape(-1, in_features)
    batch = x2.shape[0]
    bias2 = bias.reshape(1, out_features)
    packed = mask.astype(jnp.int4)
    y = pl.pallas_call(
        _body,
        grid=(out_features // OUT_BLK,),
        in_specs=[
            pl.BlockSpec((batch, in_features), lambda j: (0, 0)),
            pl.BlockSpec((OUT_BLK, in_features), lambda j: (j, 0)),
            pl.BlockSpec((OUT_BLK, in_features), lambda j: (j, 0)),
            pl.BlockSpec((1, OUT_BLK), lambda j: (0, j)),
        ],
        out_specs=pl.BlockSpec((batch, OUT_BLK), lambda j: (0, j)),
        out_shape=jax.ShapeDtypeStruct((batch, out_features), jnp.float32),
        compiler_params=pltpu.CompilerParams(
            allow_input_fusion=[False, False, True, False],
        ),
    )(x2, W, packed, bias2)
    return y.reshape(orig_shape[:-1] + (out_features,))
